# baseline (speedup 1.0000x reference)
import functools

import jax
import jax.numpy as jnp
from jax import lax
from jax.experimental import pallas as pl
from jax.experimental.pallas import tpu as pltpu
from jax.experimental.pallas import tpu_sc as plsc

N_NODES = 10000
N_EDGES = 320000
D = 128

NUM_CORES = 2
NUM_SUBCORES = 16
NUM_TILES = NUM_CORES * NUM_SUBCORES
CHUNK = 128
K_CHUNKS = 80
BLOCK = 8
E_PAD = NUM_TILES * CHUNK * K_CHUNKS
N_PAD = 10240
ROWS_PER_SUBCORE = N_PAD // NUM_SUBCORES
LANES = 16


def _spmm_body(x_hbm, src_hbm, dst_hbm, w_hbm, out_hbm,
               src_v, dst_blk, w_blk, rows_a, rows_b, acc_sh, sem_a, sem_b):
  c = lax.axis_index("c")
  s = lax.axis_index("s")
  tid = c * NUM_SUBCORES + s

  pltpu.sync_copy(src_hbm.at[tid], src_v)

  zero = jnp.zeros((LANES,), jnp.float32)

  def _zero_row(r, carry):
    for j in range(D // LANES):
      rows_a[r, pl.ds(j * LANES, LANES)] = zero
    return carry

  lax.fori_loop(0, CHUNK, _zero_row, 0)
  row0 = s * ROWS_PER_SUBCORE
  for i in range(ROWS_PER_SUBCORE // CHUNK):
    pltpu.sync_copy(rows_a, acc_sh.at[pl.ds(row0 + i * CHUNK, CHUNK)])
  plsc.subcore_barrier()

  def _scale(rows_ref, j):
    def _scale_group(g, inner):
      wvec = w_blk[j, pl.ds(g * LANES, LANES)]
      for r in range(LANES):
        wb = jnp.full((LANES,), wvec[r])
        row = g * LANES + r
        for jj in range(D // LANES):
          sl = pl.ds(jj * LANES, LANES)
          rows_ref[row, sl] = rows_ref[row, sl] * wb
      return inner

    lax.fori_loop(0, CHUNK // LANES, _scale_group, 0)

  def _block(m, carry):
    k0 = m * BLOCK
    pltpu.sync_copy(dst_hbm.at[tid].at[pl.ds(k0, BLOCK)], dst_blk)
    pltpu.sync_copy(w_hbm.at[tid].at[pl.ds(k0, BLOCK)], w_blk)
    for p in range(BLOCK // 2):
      ja = 2 * p
      jb = ja + 1
      desc_a = pltpu.async_copy(x_hbm.at[src_v.at[k0 + ja]], rows_a, sem_a)
      desc_b = pltpu.async_copy(x_hbm.at[src_v.at[k0 + jb]], rows_b, sem_b)
      desc_a.wait()
      _scale(rows_a, ja)
      pltpu.sync_copy(rows_a, acc_sh.at[dst_blk.at[ja]], add=True)
      desc_b.wait()
      _scale(rows_b, jb)
      pltpu.sync_copy(rows_b, acc_sh.at[dst_blk.at[jb]], add=True)
    return carry

  lax.fori_loop(0, K_CHUNKS // BLOCK, _block, 0)
  plsc.subcore_barrier()

  pltpu.sync_copy(acc_sh.at[pl.ds(row0, ROWS_PER_SUBCORE)],
                  out_hbm.at[c].at[pl.ds(row0, ROWS_PER_SUBCORE)])


_spmm_sc = pl.kernel(
    _spmm_body,
    out_type=jax.ShapeDtypeStruct((NUM_CORES, N_PAD, D), jnp.float32),
    mesh=plsc.VectorSubcoreMesh(core_axis_name="c", subcore_axis_name="s"),
    scratch_types=[
        pltpu.VMEM((K_CHUNKS, CHUNK), jnp.int32),
        pltpu.VMEM((BLOCK, CHUNK), jnp.int32),
        pltpu.VMEM((BLOCK, CHUNK), jnp.float32),
        pltpu.VMEM((CHUNK, D), jnp.float32),
        pltpu.VMEM((CHUNK, D), jnp.float32),
        pltpu.VMEM_SHARED((N_PAD, D), jnp.float32),
        pltpu.SemaphoreType.DMA,
        pltpu.SemaphoreType.DMA,
    ],
    name="spmm_sc",
)


def _proj_body(p_ref, w_ref, o_ref):
  h = p_ref[0] + p_ref[1]
  o_ref[...] = jnp.dot(h, w_ref[...], preferred_element_type=jnp.float32)


def _relu_body(p_ref, o_ref):
  o_ref[...] = jnp.maximum(p_ref[0] + p_ref[1], 0.0)


_BM = 1024


def _proj_tc(p, W):
  return pl.pallas_call(
      _proj_body,
      out_shape=jax.ShapeDtypeStruct((N_PAD, D), jnp.float32),
      grid=(N_PAD // _BM,),
      in_specs=[
          pl.BlockSpec((NUM_CORES, _BM, D), lambda i: (0, i, 0)),
          pl.BlockSpec((D, D), lambda i: (0, 0)),
      ],
      out_specs=pl.BlockSpec((_BM, D), lambda i: (i, 0)),
  )(p, W)


def _relu_tc(p):
  return pl.pallas_call(
      _relu_body,
      out_shape=jax.ShapeDtypeStruct((N_PAD, D), jnp.float32),
      grid=(N_PAD // _BM,),
      in_specs=[pl.BlockSpec((NUM_CORES, _BM, D), lambda i: (0, i, 0))],
      out_specs=pl.BlockSpec((_BM, D), lambda i: (i, 0)),
  )(p)


def kernel(x, edge_index, edge_weight, W):
  src = edge_index[1].astype(jnp.int32)
  dst = edge_index[0].astype(jnp.int32)
  w = edge_weight.astype(jnp.float32)
  pad = E_PAD - N_EDGES
  src = jnp.pad(src, (0, pad)).reshape(NUM_TILES, K_CHUNKS, CHUNK)
  dst = jnp.pad(dst, (0, pad)).reshape(NUM_TILES, K_CHUNKS, CHUNK)
  w = jnp.pad(w, (0, pad)).reshape(NUM_TILES, K_CHUNKS, CHUNK)

  p1 = _spmm_sc(x.astype(jnp.float32), src, dst, w)
  y = _proj_tc(p1, W.astype(jnp.float32))
  p2 = _spmm_sc(y[:N_NODES], src, dst, w)
  return _relu_tc(p2)[:N_NODES]


# speedup vs baseline: 1.4776x; 1.4776x over previous
import functools

import jax
import jax.numpy as jnp
from jax import lax
from jax.experimental import pallas as pl
from jax.experimental.pallas import tpu as pltpu
from jax.experimental.pallas import tpu_sc as plsc

N_NODES = 10000
N_EDGES = 320000
D = 128

NUM_CORES = 2
NUM_SUBCORES = 16
NUM_TILES = NUM_CORES * NUM_SUBCORES
CHUNK = 128
K_CHUNKS = -(-N_EDGES // (NUM_TILES * CHUNK))
E_PAD = NUM_TILES * CHUNK * K_CHUNKS
N_PAD = 10240
ROWS_PER_SUBCORE = N_PAD // NUM_SUBCORES
LANES = 16


def _spmm_body(x_hbm, src_hbm, dst_hbm, w_hbm, out_hbm,
               src_v, dst_v, w_v, rows_v, acc_sh, gsem):
  c = lax.axis_index("c")
  s = lax.axis_index("s")
  tid = c * NUM_SUBCORES + s

  pltpu.sync_copy(src_hbm.at[tid], src_v)
  pltpu.sync_copy(dst_hbm.at[tid], dst_v)
  pltpu.sync_copy(w_hbm.at[tid], w_v)

  zero = jnp.zeros((LANES,), jnp.float32)

  def _zero_row(r, carry):
    for j in range(D // LANES):
      rows_v[r, pl.ds(j * LANES, LANES)] = zero
    return carry

  lax.fori_loop(0, CHUNK, _zero_row, 0)
  row0 = s * ROWS_PER_SUBCORE
  for i in range(ROWS_PER_SUBCORE // CHUNK):
    pltpu.sync_copy(rows_v, acc_sh.at[pl.ds(row0 + i * CHUNK, CHUNK)])
  plsc.subcore_barrier()

  def _chunk(k, carry):
    pltpu.async_copy(x_hbm.at[src_v.at[k]], rows_v, gsem).wait()

    def _scale_group(g, inner):
      wvec = w_v[pl.ds(k * CHUNK + g * LANES, LANES)]
      for r in range(LANES):
        wb = jnp.full((LANES,), wvec[r])
        row = g * LANES + r
        for j in range(D // LANES):
          sl = pl.ds(j * LANES, LANES)
          rows_v[row, sl] = rows_v[row, sl] * wb
      return inner

    lax.fori_loop(0, CHUNK // LANES, _scale_group, 0)
    pltpu.sync_copy(rows_v, acc_sh.at[dst_v.at[k]], add=True)
    return carry

  lax.fori_loop(0, K_CHUNKS, _chunk, 0)
  plsc.subcore_barrier()

  pltpu.sync_copy(acc_sh.at[pl.ds(row0, ROWS_PER_SUBCORE)],
                  out_hbm.at[c].at[pl.ds(row0, ROWS_PER_SUBCORE)])


_spmm_sc = pl.kernel(
    _spmm_body,
    out_type=jax.ShapeDtypeStruct((NUM_CORES, N_PAD, D), jnp.float32),
    mesh=plsc.VectorSubcoreMesh(core_axis_name="c", subcore_axis_name="s"),
    scratch_types=[
        pltpu.VMEM((K_CHUNKS, CHUNK), jnp.int32),
        pltpu.VMEM((K_CHUNKS, CHUNK), jnp.int32),
        pltpu.VMEM((K_CHUNKS * CHUNK,), jnp.float32),
        pltpu.VMEM((CHUNK, D), jnp.float32),
        pltpu.VMEM_SHARED((N_PAD, D), jnp.float32),
        pltpu.SemaphoreType.DMA,
    ],
    name="spmm_sc",
)


def _proj_body(p_ref, w_ref, o_ref):
  h = p_ref[0] + p_ref[1]
  o_ref[...] = jnp.dot(h, w_ref[...], preferred_element_type=jnp.float32)


def _relu_body(p_ref, o_ref):
  o_ref[...] = jnp.maximum(p_ref[0] + p_ref[1], 0.0)


_BM = 1024


def _proj_tc(p, W):
  return pl.pallas_call(
      _proj_body,
      out_shape=jax.ShapeDtypeStruct((N_PAD, D), jnp.float32),
      grid=(N_PAD // _BM,),
      in_specs=[
          pl.BlockSpec((NUM_CORES, _BM, D), lambda i: (0, i, 0)),
          pl.BlockSpec((D, D), lambda i: (0, 0)),
      ],
      out_specs=pl.BlockSpec((_BM, D), lambda i: (i, 0)),
  )(p, W)


def _relu_tc(p):
  return pl.pallas_call(
      _relu_body,
      out_shape=jax.ShapeDtypeStruct((N_PAD, D), jnp.float32),
      grid=(N_PAD // _BM,),
      in_specs=[pl.BlockSpec((NUM_CORES, _BM, D), lambda i: (0, i, 0))],
      out_specs=pl.BlockSpec((_BM, D), lambda i: (i, 0)),
  )(p)


def kernel(x, edge_index, edge_weight, W):
  src = edge_index[1].astype(jnp.int32)
  dst = edge_index[0].astype(jnp.int32)
  w = edge_weight.astype(jnp.float32)
  pad = E_PAD - N_EDGES
  src = jnp.pad(src, (0, pad)).reshape(NUM_TILES, K_CHUNKS, CHUNK)
  dst = jnp.pad(dst, (0, pad)).reshape(NUM_TILES, K_CHUNKS, CHUNK)
  w = jnp.pad(w, (0, pad)).reshape(NUM_TILES, K_CHUNKS * CHUNK)

  p1 = _spmm_sc(x.astype(jnp.float32), src, dst, w)
  y = _proj_tc(p1, W.astype(jnp.float32))
  p2 = _spmm_sc(y[:N_NODES], src, dst, w)
  return _relu_tc(p2)[:N_NODES]


# speedup vs baseline: 1.4820x; 1.0029x over previous
import functools

import jax
import jax.numpy as jnp
from jax import lax
from jax.experimental import pallas as pl
from jax.experimental.pallas import tpu as pltpu
from jax.experimental.pallas import tpu_sc as plsc

N_NODES = 10000
N_EDGES = 320000
D = 128

NUM_CORES = 2
NUM_SUBCORES = 16
NUM_TILES = NUM_CORES * NUM_SUBCORES
CHUNK = 128
K_CHUNKS = -(-N_EDGES // (NUM_TILES * CHUNK))
E_PAD = NUM_TILES * CHUNK * K_CHUNKS
N_PAD = 10240
ROWS_PER_SUBCORE = N_PAD // NUM_SUBCORES
LANES = 16


def _spmm_body(x_hbm, src_hbm, dst_hbm, w_hbm, out_hbm,
               src_v, dst_v, w_v, rows_v, acc_sh, gsem):
  c = lax.axis_index("c")
  s = lax.axis_index("s")
  tid = c * NUM_SUBCORES + s

  pltpu.sync_copy(src_hbm.at[tid], src_v)
  pltpu.sync_copy(dst_hbm.at[tid], dst_v)
  pltpu.sync_copy(w_hbm.at[tid], w_v)

  zero = jnp.zeros((LANES,), jnp.float32)

  def _zero_row(r, carry):
    for j in range(D // LANES):
      rows_v[r, pl.ds(j * LANES, LANES)] = zero
    return carry

  lax.fori_loop(0, CHUNK, _zero_row, 0)
  row0 = s * ROWS_PER_SUBCORE
  for i in range(ROWS_PER_SUBCORE // CHUNK):
    pltpu.sync_copy(rows_v, acc_sh.at[pl.ds(row0 + i * CHUNK, CHUNK)])
  plsc.subcore_barrier()

  off = (tid * K_CHUNKS) // NUM_TILES

  def _chunk(k0, carry):
    k = lax.rem(k0 + off, K_CHUNKS)
    pltpu.async_copy(x_hbm.at[src_v.at[k]], rows_v, gsem).wait()

    def _scale_group(g, inner):
      wvec = w_v[pl.ds(k * CHUNK + g * LANES, LANES)]
      for r in range(LANES):
        wb = jnp.full((LANES,), wvec[r])
        row = g * LANES + r
        for j in range(D // LANES):
          sl = pl.ds(j * LANES, LANES)
          rows_v[row, sl] = rows_v[row, sl] * wb
      return inner

    lax.fori_loop(0, CHUNK // LANES, _scale_group, 0)
    pltpu.sync_copy(rows_v, acc_sh.at[dst_v.at[k]], add=True)
    return carry

  lax.fori_loop(0, K_CHUNKS, _chunk, 0)
  plsc.subcore_barrier()

  pltpu.sync_copy(acc_sh.at[pl.ds(row0, ROWS_PER_SUBCORE)],
                  out_hbm.at[c].at[pl.ds(row0, ROWS_PER_SUBCORE)])


_spmm_sc = pl.kernel(
    _spmm_body,
    out_type=jax.ShapeDtypeStruct((NUM_CORES, N_PAD, D), jnp.float32),
    mesh=plsc.VectorSubcoreMesh(core_axis_name="c", subcore_axis_name="s"),
    scratch_types=[
        pltpu.VMEM((K_CHUNKS, CHUNK), jnp.int32),
        pltpu.VMEM((K_CHUNKS, CHUNK), jnp.int32),
        pltpu.VMEM((K_CHUNKS * CHUNK,), jnp.float32),
        pltpu.VMEM((CHUNK, D), jnp.float32),
        pltpu.VMEM_SHARED((N_PAD, D), jnp.float32),
        pltpu.SemaphoreType.DMA,
    ],
    name="spmm_sc",
)


def _proj_body(p_ref, w_ref, o_ref):
  h = p_ref[0] + p_ref[1]
  o_ref[...] = jnp.dot(h, w_ref[...], preferred_element_type=jnp.float32)


def _relu_body(p_ref, o_ref):
  o_ref[...] = jnp.maximum(p_ref[0] + p_ref[1], 0.0)


_BM = 1024


def _proj_tc(p, W):
  return pl.pallas_call(
      _proj_body,
      out_shape=jax.ShapeDtypeStruct((N_PAD, D), jnp.float32),
      grid=(N_PAD // _BM,),
      in_specs=[
          pl.BlockSpec((NUM_CORES, _BM, D), lambda i: (0, i, 0)),
          pl.BlockSpec((D, D), lambda i: (0, 0)),
      ],
      out_specs=pl.BlockSpec((_BM, D), lambda i: (i, 0)),
  )(p, W)


def _relu_tc(p):
  return pl.pallas_call(
      _relu_body,
      out_shape=jax.ShapeDtypeStruct((N_PAD, D), jnp.float32),
      grid=(N_PAD // _BM,),
      in_specs=[pl.BlockSpec((NUM_CORES, _BM, D), lambda i: (0, i, 0))],
      out_specs=pl.BlockSpec((_BM, D), lambda i: (i, 0)),
  )(p)


def kernel(x, edge_index, edge_weight, W):
  src = edge_index[1].astype(jnp.int32)
  dst = edge_index[0].astype(jnp.int32)
  w = edge_weight.astype(jnp.float32)
  pad = E_PAD - N_EDGES
  src = jnp.pad(src, (0, pad)).reshape(NUM_TILES, K_CHUNKS, CHUNK)
  dst = jnp.pad(dst, (0, pad)).reshape(NUM_TILES, K_CHUNKS, CHUNK)
  w = jnp.pad(w, (0, pad)).reshape(NUM_TILES, K_CHUNKS * CHUNK)

  p1 = _spmm_sc(x.astype(jnp.float32), src, dst, w)
  y = _proj_tc(p1, W.astype(jnp.float32))
  p2 = _spmm_sc(y[:N_NODES], src, dst, w)
  return _relu_tc(p2)[:N_NODES]


# speedup vs baseline: 1.4822x; 1.0001x over previous
import jax
import jax.numpy as jnp
from jax import lax
from jax.experimental import pallas as pl
from jax.experimental.pallas import tpu as pltpu
from jax.experimental.pallas import tpu_sc as plsc

N_NODES = 10000
N_EDGES = 320000
D = 128

NUM_CORES = 2
NUM_SUBCORES = 16
NUM_TILES = NUM_CORES * NUM_SUBCORES
CHUNK = 128
K_CHUNKS = -(-N_EDGES // (NUM_TILES * CHUNK))
E_PAD = NUM_TILES * CHUNK * K_CHUNKS
N_PAD = 10240
ROWS_PER_SUBCORE = N_PAD // NUM_SUBCORES
LANES = 16


def _spmm_body(x_hbm, src_hbm, dst_hbm, w_hbm, out_hbm,
               src_v, dst_v, w_v, rows_v, acc_sh, gsem):
  c = lax.axis_index("c")
  s = lax.axis_index("s")
  tid = c * NUM_SUBCORES + s

  pltpu.sync_copy(src_hbm.at[tid], src_v)
  pltpu.sync_copy(dst_hbm.at[tid], dst_v)
  pltpu.sync_copy(w_hbm.at[tid], w_v)

  zero = jnp.zeros((LANES,), jnp.float32)

  def _zero_row(r, carry):
    for j in range(D // LANES):
      rows_v[r, pl.ds(j * LANES, LANES)] = zero
    return carry

  lax.fori_loop(0, CHUNK, _zero_row, 0)
  row0 = s * ROWS_PER_SUBCORE
  for i in range(ROWS_PER_SUBCORE // CHUNK):
    pltpu.sync_copy(rows_v, acc_sh.at[pl.ds(row0 + i * CHUNK, CHUNK)])
  plsc.subcore_barrier()

  off = (tid * K_CHUNKS) // NUM_TILES

  def _chunk(k0, carry):
    k = lax.rem(k0 + off, K_CHUNKS)
    pltpu.async_copy(x_hbm.at[src_v.at[k]], rows_v, gsem).wait()

    def _scale_group(g, inner):
      wvec = w_v[pl.ds(k * CHUNK + g * LANES, LANES)]
      for r in range(LANES):
        wb = jnp.full((LANES,), wvec[r])
        row = g * LANES + r
        for j in range(D // LANES):
          sl = pl.ds(j * LANES, LANES)
          rows_v[row, sl] = rows_v[row, sl] * wb
      return inner

    lax.fori_loop(0, CHUNK // LANES, _scale_group, 0)
    pltpu.sync_copy(rows_v, acc_sh.at[dst_v.at[k]], add=True)
    return carry

  lax.fori_loop(0, K_CHUNKS, _chunk, 0)
  plsc.subcore_barrier()

  pltpu.sync_copy(acc_sh.at[pl.ds(row0, ROWS_PER_SUBCORE)],
                  out_hbm.at[c].at[pl.ds(row0, ROWS_PER_SUBCORE)])


_spmm_sc = pl.kernel(
    _spmm_body,
    out_type=jax.ShapeDtypeStruct((NUM_CORES, N_PAD, D), jnp.float32),
    mesh=plsc.VectorSubcoreMesh(core_axis_name="c", subcore_axis_name="s"),
    scratch_types=[
        pltpu.VMEM((K_CHUNKS, CHUNK), jnp.int32),
        pltpu.VMEM((K_CHUNKS, CHUNK), jnp.int32),
        pltpu.VMEM((K_CHUNKS * CHUNK,), jnp.float32),
        pltpu.VMEM((CHUNK, D), jnp.float32),
        pltpu.VMEM_SHARED((N_PAD, D), jnp.float32),
        pltpu.SemaphoreType.DMA,
    ],
    name="spmm_sc",
)


def _proj_body(p_ref, w_ref, o_ref):
  h = p_ref[0] + p_ref[1]
  o_ref[...] = jnp.dot(h, w_ref[...], preferred_element_type=jnp.float32)


def _relu_body(p_ref, o_ref):
  o_ref[...] = jnp.maximum(p_ref[0] + p_ref[1], 0.0)


_BM = 1024


def _proj_tc(p, W):
  return pl.pallas_call(
      _proj_body,
      out_shape=jax.ShapeDtypeStruct((N_PAD, D), jnp.float32),
      grid=(N_PAD // _BM,),
      in_specs=[
          pl.BlockSpec((NUM_CORES, _BM, D), lambda i: (0, i, 0)),
          pl.BlockSpec((D, D), lambda i: (0, 0)),
      ],
      out_specs=pl.BlockSpec((_BM, D), lambda i: (i, 0)),
  )(p, W)


def _relu_tc(p):
  return pl.pallas_call(
      _relu_body,
      out_shape=jax.ShapeDtypeStruct((N_PAD, D), jnp.float32),
      grid=(N_PAD // _BM,),
      in_specs=[pl.BlockSpec((NUM_CORES, _BM, D), lambda i: (0, i, 0))],
      out_specs=pl.BlockSpec((_BM, D), lambda i: (i, 0)),
  )(p)


def kernel(x, edge_index, edge_weight, W):
  src = edge_index[1].astype(jnp.int32)
  dst = edge_index[0].astype(jnp.int32)
  w = edge_weight.astype(jnp.float32)
  pad = E_PAD - N_EDGES
  src = jnp.pad(src, (0, pad)).reshape(NUM_TILES, K_CHUNKS, CHUNK)
  dst = jnp.pad(dst, (0, pad)).reshape(NUM_TILES, K_CHUNKS, CHUNK)
  w = jnp.pad(w, (0, pad)).reshape(NUM_TILES, K_CHUNKS * CHUNK)

  p1 = _spmm_sc(x.astype(jnp.float32), src, dst, w)
  y = _proj_tc(p1, W.astype(jnp.float32))
  p2 = _spmm_sc(y[:N_NODES], src, dst, w)
  return _relu_tc(p2)[:N_NODES]
